# Initial kernel scaffold; baseline (speedup 1.0000x reference)
import jax
import jax.numpy as jnp
from jax.experimental import pallas as pl


def kernel(x, row_emb, col_emb):
    raise NotImplementedError("write your pallas kernel here")


# speedup vs baseline: 1.0191x; 1.0191x over previous
import jax
import jax.numpy as jnp
from jax.experimental import pallas as pl
from jax.experimental.pallas import tpu as pltpu


def _add_bias_kernel(x_ref, r_ref, c_ref, o_ref):
    r = r_ref[...]
    c = c_ref[...]
    o_ref[...] = x_ref[...] + (r[None, :, None, :] + c[None, None, :, :])


def kernel(x, row_emb, col_emb):
    n, s, d = x.shape
    x4 = x.reshape(n, 8, 8, d)
    blk = 256
    grid = (n // blk,)
    out = pl.pallas_call(
        _add_bias_kernel,
        grid=grid,
        in_specs=[
            pl.BlockSpec((blk, 8, 8, d), lambda i: (i, 0, 0, 0)),
            pl.BlockSpec((8, d), lambda i: (0, 0)),
            pl.BlockSpec((8, d), lambda i: (0, 0)),
        ],
        out_specs=pl.BlockSpec((blk, 8, 8, d), lambda i: (i, 0, 0, 0)),
        out_shape=jax.ShapeDtypeStruct((n, 8, 8, d), x.dtype),
        compiler_params=pltpu.CompilerParams(
            dimension_semantics=("parallel",),
        ),
    )(x4, row_emb, col_emb)
    return out.reshape(n, s, d)
